# baseline (speedup 1.0000x reference)
import functools

import numpy as np

import jax
import jax.numpy as jnp
from jax.experimental import pallas as pl
from jax.experimental.pallas import tpu as pltpu

ALPHA = 0.05
EPS = 1e-5
TM = 256
NBUF = 4


def _slab_copy(adj_hbm, buf_ref, sem, j, slot, tm):
    return pltpu.make_async_copy(
        adj_hbm.at[pl.ds(j * tm, tm), :],
        buf_ref.at[pl.ds(slot * tm, tm), :],
        sem.at[slot],
    )


def _fused_kernel(x_hbm, adj_hbm, w_ref, gamma_ref, beta_ref, mask_ref,
                  out_ref, x2d_ref, y_ref, buf_ref, z_ref, xsem, asem,
                  *, n_mod, d, nsteps):
    i = pl.program_id(0)
    tm = TM

    @pl.when(i == 0)
    def _prologue():
        for m in range(n_mod):
            pltpu.make_async_copy(
                x_hbm.at[:, m, :], x2d_ref.at[:, m * d:(m + 1) * d], xsem
            ).start()
        for j in range(NBUF - 1):
            _slab_copy(adj_hbm, buf_ref, asem, j, j, tm).start()
        for m in range(n_mod):
            pltpu.make_async_copy(
                x_hbm.at[:, m, :], x2d_ref.at[:, m * d:(m + 1) * d], xsem
            ).wait()
        w = w_ref[...].astype(jnp.bfloat16)
        for m in range(n_mod):
            sl = slice(m * d, (m + 1) * d)
            xm = x2d_ref[:, sl].astype(jnp.bfloat16)
            y_ref[:, sl] = jnp.dot(
                xm, w, preferred_element_type=jnp.float32).astype(jnp.bfloat16)

    nxt = i + NBUF - 1

    @pl.when(nxt < nsteps)
    def _prefetch():
        for c in range(NBUF):
            @pl.when(jax.lax.rem(nxt, NBUF) == c)
            def _go():
                _slab_copy(adj_hbm, buf_ref, asem, nxt, c, tm).start()

    @pl.when(i > 0)
    def _epilogue():
        zoff = ((i - 1) % 2) * tm
        z = z_ref[pl.ds(zoff, tm), :]
        v = x2d_ref[pl.ds((i - 1) * tm, tm), :] + ALPHA * z
        vb = v.astype(jnp.bfloat16)
        v2b = (v * v).astype(jnp.bfloat16)
        stat = jnp.dot(jnp.concatenate([vb, v2b], axis=0), mask_ref[...],
                       preferred_element_type=jnp.float32)
        mu = stat[:tm, :]
        var = stat[tm:, :] - mu * mu
        s = jax.lax.rsqrt(var + EPS)
        o = (v - mu) * s * gamma_ref[...] + beta_ref[...]
        for m in range(n_mod):
            out_ref[:, m, :] = o[:, m * d:(m + 1) * d]

    @pl.when(i < nsteps)
    def _gemm():
        zoff = (i % 2) * tm
        for c in range(NBUF):
            @pl.when(jax.lax.rem(i, NBUF) == c)
            def _consume():
                _slab_copy(adj_hbm, buf_ref, asem, i, c, tm).wait()
                adj = buf_ref[pl.ds(c * tm, tm), :].astype(jnp.bfloat16)
                z_ref[pl.ds(zoff, tm), :] = jnp.dot(
                    adj, y_ref[...], preferred_element_type=jnp.float32)


@jax.jit
def kernel(multimodal, adj, W, gamma, beta):
    n, n_mod, d = multimodal.shape
    md = n_mod * d
    nsteps = n // TM
    gamma2 = jnp.tile(gamma, n_mod).reshape(1, md)
    beta2 = jnp.tile(beta, n_mod).reshape(1, md)
    mask = np.kron(np.eye(n_mod, dtype=np.float32),
                   np.full((d, d), 1.0 / d, dtype=np.float32))
    mask = jnp.asarray(mask, dtype=jnp.bfloat16)
    out = pl.pallas_call(
        functools.partial(_fused_kernel, n_mod=n_mod, d=d, nsteps=nsteps),
        grid=(nsteps + 1,),
        in_specs=[
            pl.BlockSpec(memory_space=pl.ANY),
            pl.BlockSpec(memory_space=pl.ANY),
            pl.BlockSpec((d, d), lambda i: (0, 0)),
            pl.BlockSpec((1, md), lambda i: (0, 0)),
            pl.BlockSpec((1, md), lambda i: (0, 0)),
            pl.BlockSpec((md, md), lambda i: (0, 0)),
        ],
        out_specs=pl.BlockSpec((TM, n_mod, d),
                               lambda i: (jnp.maximum(i - 1, 0), 0, 0)),
        out_shape=jax.ShapeDtypeStruct((n, n_mod, d), jnp.float32),
        scratch_shapes=[
            pltpu.VMEM((n, md), jnp.float32),
            pltpu.VMEM((n, md), jnp.bfloat16),
            pltpu.VMEM((NBUF * TM, n), jnp.float32),
            pltpu.VMEM((2 * TM, md), jnp.float32),
            pltpu.SemaphoreType.DMA,
            pltpu.SemaphoreType.DMA((NBUF,)),
        ],
        compiler_params=pltpu.CompilerParams(
            dimension_semantics=("arbitrary",),
        ),
    )(multimodal, adj, W, gamma2, beta2, mask)
    return out


# speedup vs baseline: 1.0586x; 1.0586x over previous
import functools

import numpy as np

import jax
import jax.numpy as jnp
from jax.experimental import pallas as pl
from jax.experimental.pallas import tpu as pltpu

ALPHA = 0.05
EPS = 1e-5
TM = 512
NBUF = 3


def _slab_copy(adj_hbm, buf_ref, sem, j, slot, tm):
    return pltpu.make_async_copy(
        adj_hbm.at[pl.ds(j * tm, tm), :],
        buf_ref.at[pl.ds(slot * tm, tm), :],
        sem.at[slot],
    )


def _fused_kernel(x_hbm, adj_hbm, w_ref, gamma_ref, beta_ref, mask_ref,
                  out_ref, x2d_ref, y_ref, buf_ref, z_ref, xsem, asem,
                  *, n_mod, d, nsteps):
    i = pl.program_id(0)
    tm = TM

    @pl.when(i == 0)
    def _prologue():
        for m in range(n_mod):
            pltpu.make_async_copy(
                x_hbm.at[:, m, :], x2d_ref.at[:, m * d:(m + 1) * d], xsem
            ).start()
        for j in range(NBUF - 1):
            _slab_copy(adj_hbm, buf_ref, asem, j, j, tm).start()
        for m in range(n_mod):
            pltpu.make_async_copy(
                x_hbm.at[:, m, :], x2d_ref.at[:, m * d:(m + 1) * d], xsem
            ).wait()
        w = w_ref[...].astype(jnp.bfloat16)
        for m in range(n_mod):
            sl = slice(m * d, (m + 1) * d)
            xm = x2d_ref[:, sl].astype(jnp.bfloat16)
            y_ref[:, sl] = jnp.dot(
                xm, w, preferred_element_type=jnp.float32).astype(jnp.bfloat16)

    nxt = i + NBUF - 1

    @pl.when(nxt < nsteps)
    def _prefetch():
        for c in range(NBUF):
            @pl.when(jax.lax.rem(nxt, NBUF) == c)
            def _go():
                _slab_copy(adj_hbm, buf_ref, asem, nxt, c, tm).start()

    @pl.when(i > 0)
    def _epilogue():
        zoff = ((i - 1) % 2) * tm
        z = z_ref[pl.ds(zoff, tm), :]
        v = x2d_ref[pl.ds((i - 1) * tm, tm), :] + ALPHA * z
        vb = v.astype(jnp.bfloat16)
        v2b = vb * vb
        stat = jnp.dot(jnp.concatenate([vb, v2b], axis=0), mask_ref[...],
                       preferred_element_type=jnp.float32)
        mu = stat[:tm, :]
        var = stat[tm:, :] - mu * mu
        s = jax.lax.rsqrt(var + EPS)
        o = (v - mu) * s * gamma_ref[...] + beta_ref[...]
        for m in range(n_mod):
            out_ref[:, m, :] = o[:, m * d:(m + 1) * d]

    @pl.when(i < nsteps)
    def _gemm():
        zoff = (i % 2) * tm
        for c in range(NBUF):
            @pl.when(jax.lax.rem(i, NBUF) == c)
            def _consume():
                _slab_copy(adj_hbm, buf_ref, asem, i, c, tm).wait()
                adj = buf_ref[pl.ds(c * tm, tm), :].astype(jnp.bfloat16)
                z_ref[pl.ds(zoff, tm), :] = jnp.dot(
                    adj, y_ref[...], preferred_element_type=jnp.float32)


@jax.jit
def kernel(multimodal, adj, W, gamma, beta):
    n, n_mod, d = multimodal.shape
    md = n_mod * d
    nsteps = n // TM
    gamma2 = jnp.tile(gamma, n_mod).reshape(1, md)
    beta2 = jnp.tile(beta, n_mod).reshape(1, md)
    mask = np.kron(np.eye(n_mod, dtype=np.float32),
                   np.full((d, d), 1.0 / d, dtype=np.float32))
    mask = jnp.asarray(mask, dtype=jnp.bfloat16)
    out = pl.pallas_call(
        functools.partial(_fused_kernel, n_mod=n_mod, d=d, nsteps=nsteps),
        grid=(nsteps + 1,),
        in_specs=[
            pl.BlockSpec(memory_space=pl.ANY),
            pl.BlockSpec(memory_space=pl.ANY),
            pl.BlockSpec((d, d), lambda i: (0, 0)),
            pl.BlockSpec((1, md), lambda i: (0, 0)),
            pl.BlockSpec((1, md), lambda i: (0, 0)),
            pl.BlockSpec((md, md), lambda i: (0, 0)),
        ],
        out_specs=pl.BlockSpec((TM, n_mod, d),
                               lambda i: (jnp.maximum(i - 1, 0), 0, 0)),
        out_shape=jax.ShapeDtypeStruct((n, n_mod, d), jnp.float32),
        scratch_shapes=[
            pltpu.VMEM((n, md), jnp.float32),
            pltpu.VMEM((n, md), jnp.bfloat16),
            pltpu.VMEM((NBUF * TM, n), jnp.float32),
            pltpu.VMEM((2 * TM, md), jnp.float32),
            pltpu.SemaphoreType.DMA,
            pltpu.SemaphoreType.DMA((NBUF,)),
        ],
        compiler_params=pltpu.CompilerParams(
            dimension_semantics=("arbitrary",),
        ),
    )(multimodal, adj, W, gamma2, beta2, mask)
    return out


# speedup vs baseline: 1.1650x; 1.1005x over previous
import functools

import numpy as np

import jax
import jax.numpy as jnp
from jax.experimental import pallas as pl
from jax.experimental.pallas import tpu as pltpu

ALPHA = 0.05
EPS = 1e-5
TM = 512
NBUF = 3


def _slab_copy(adj_hbm, buf_ref, sem, j, slot, tm):
    return pltpu.make_async_copy(
        adj_hbm.at[pl.ds(j * tm, tm), :],
        buf_ref.at[pl.ds(slot * tm, tm), :],
        sem.at[slot],
    )


def _out_copy(o2d_ref, out_hbm, sem, t, slot, m, tm, d):
    return pltpu.make_async_copy(
        o2d_ref.at[pl.ds(slot * tm, tm), m * d:(m + 1) * d],
        out_hbm.at[pl.ds(t * tm, tm), m, :],
        sem.at[slot],
    )


def _fused_kernel(x_hbm, adj_hbm, w_ref, gamma_ref, beta_ref, mask_ref,
                  out_hbm, x2d_ref, y_ref, buf_ref, z_ref, o2d_ref,
                  xsem, asem, osem, *, n_mod, d, nsteps):
    i = pl.program_id(0)
    tm = TM

    @pl.when(i == 0)
    def _prologue():
        for m in range(n_mod):
            pltpu.make_async_copy(
                x_hbm.at[:, m, :], x2d_ref.at[:, m * d:(m + 1) * d], xsem
            ).start()
        for j in range(NBUF - 1):
            _slab_copy(adj_hbm, buf_ref, asem, j, j, tm).start()
        for m in range(n_mod):
            pltpu.make_async_copy(
                x_hbm.at[:, m, :], x2d_ref.at[:, m * d:(m + 1) * d], xsem
            ).wait()
        w = w_ref[...].astype(jnp.bfloat16)
        for m in range(n_mod):
            sl = slice(m * d, (m + 1) * d)
            xm = x2d_ref[:, sl].astype(jnp.bfloat16)
            y_ref[:, sl] = jnp.dot(
                xm, w, preferred_element_type=jnp.float32).astype(jnp.bfloat16)

    nxt = i + NBUF - 1

    @pl.when(nxt < nsteps)
    def _prefetch():
        for c in range(NBUF):
            @pl.when(jax.lax.rem(nxt, NBUF) == c)
            def _go():
                _slab_copy(adj_hbm, buf_ref, asem, nxt, c, tm).start()

    @pl.when(i > 0)
    def _epilogue():
        t = i - 1
        for c in range(2):
            @pl.when(jax.lax.rem(t, 2) == c)
            def _epi():
                @pl.when(t >= 2)
                def _reclaim():
                    for m in range(n_mod):
                        _out_copy(o2d_ref, out_hbm, osem, t - 2, c, m,
                                  tm, d).wait()
                zoff = c * tm
                z = z_ref[pl.ds(zoff, tm), :]
                v = x2d_ref[pl.ds(t * tm, tm), :] + ALPHA * z
                vb = v.astype(jnp.bfloat16)
                v2b = vb * vb
                stat = jnp.dot(jnp.concatenate([vb, v2b], axis=0),
                               mask_ref[...],
                               preferred_element_type=jnp.float32)
                mu = stat[:tm, :]
                var = stat[tm:, :] - mu * mu
                s = jax.lax.rsqrt(var + EPS)
                o = (v - mu) * s * gamma_ref[...] + beta_ref[...]
                o2d_ref[pl.ds(c * tm, tm), :] = o
                for m in range(n_mod):
                    _out_copy(o2d_ref, out_hbm, osem, t, c, m, tm, d).start()

    @pl.when(i < nsteps)
    def _gemm():
        zoff = (i % 2) * tm
        for c in range(NBUF):
            @pl.when(jax.lax.rem(i, NBUF) == c)
            def _consume():
                _slab_copy(adj_hbm, buf_ref, asem, i, c, tm).wait()
                adj = buf_ref[pl.ds(c * tm, tm), :].astype(jnp.bfloat16)
                z_ref[pl.ds(zoff, tm), :] = jnp.dot(
                    adj, y_ref[...], preferred_element_type=jnp.float32)

    @pl.when(i == nsteps)
    def _drain():
        for t in (nsteps - 2, nsteps - 1):
            for m in range(n_mod):
                _out_copy(o2d_ref, out_hbm, osem, t, t % 2, m, tm, d).wait()


@jax.jit
def kernel(multimodal, adj, W, gamma, beta):
    n, n_mod, d = multimodal.shape
    md = n_mod * d
    nsteps = n // TM
    gamma2 = jnp.tile(gamma, n_mod).reshape(1, md)
    beta2 = jnp.tile(beta, n_mod).reshape(1, md)
    mask = np.kron(np.eye(n_mod, dtype=np.float32),
                   np.full((d, d), 1.0 / d, dtype=np.float32))
    mask = jnp.asarray(mask, dtype=jnp.bfloat16)
    out = pl.pallas_call(
        functools.partial(_fused_kernel, n_mod=n_mod, d=d, nsteps=nsteps),
        grid=(nsteps + 1,),
        in_specs=[
            pl.BlockSpec(memory_space=pl.ANY),
            pl.BlockSpec(memory_space=pl.ANY),
            pl.BlockSpec((d, d), lambda i: (0, 0)),
            pl.BlockSpec((1, md), lambda i: (0, 0)),
            pl.BlockSpec((1, md), lambda i: (0, 0)),
            pl.BlockSpec((md, md), lambda i: (0, 0)),
        ],
        out_specs=pl.BlockSpec(memory_space=pl.ANY),
        out_shape=jax.ShapeDtypeStruct((n, n_mod, d), jnp.float32),
        scratch_shapes=[
            pltpu.VMEM((n, md), jnp.float32),
            pltpu.VMEM((n, md), jnp.bfloat16),
            pltpu.VMEM((NBUF * TM, n), jnp.float32),
            pltpu.VMEM((2 * TM, md), jnp.float32),
            pltpu.VMEM((2 * TM, md), jnp.float32),
            pltpu.SemaphoreType.DMA,
            pltpu.SemaphoreType.DMA((NBUF,)),
            pltpu.SemaphoreType.DMA((2,)),
        ],
        compiler_params=pltpu.CompilerParams(
            dimension_semantics=("arbitrary",),
        ),
    )(multimodal, adj, W, gamma2, beta2, mask)
    return out
